# baseline (speedup 1.0000x reference)
import functools

import jax
import jax.numpy as jnp
from jax import lax
from jax.experimental import pallas as pl
from jax.experimental.pallas import tpu as pltpu
from jax.experimental.pallas import tpu_sc as plsc

DICT_SIZE = 100000
HIDDEN_DIM = 128
BATCH = 4096
SEQ = 50

_NC = 2
_NS = 16
_NW = _NC * _NS

_N = BATCH * SEQ
_PER_W = _N // _NW
_K = 128
_CHUNKS = _PER_W // _K
_NBUF = 5
_LOOK = 2


def _make_gather():
    mesh = plsc.VectorSubcoreMesh(
        core_axis_name="c", subcore_axis_name="s",
        num_cores=_NC, num_subcores=_NS,
    )

    @functools.partial(
        pl.kernel,
        out_type=jax.ShapeDtypeStruct((_N, HIDDEN_DIM), jnp.float32),
        mesh=mesh,
        scratch_types=[
            pltpu.VMEM((_CHUNKS, _K), jnp.int32),
            pltpu.VMEM((_NBUF, _K, HIDDEN_DIM), jnp.float32),
            [pltpu.SemaphoreType.DMA] * _NBUF,
            [pltpu.SemaphoreType.DMA] * _NBUF,
        ],
    )
    def gather_kernel(idx_hbm, table_hbm, out_hbm, idx_v, rows_v, sg, ss):
        wid = lax.axis_index("s") * _NC + lax.axis_index("c")
        base = wid * _PER_W

        def fire_gather(c, b):
            pltpu.async_copy(table_hbm.at[idx_v.at[c]], rows_v.at[b], sg[b])

        def wait_gather(c, b):
            pltpu.make_async_copy(
                table_hbm.at[idx_v.at[c]], rows_v.at[b], sg[b]
            ).wait()

        def fire_scatter(c, b):
            pltpu.async_copy(
                rows_v.at[b], out_hbm.at[pl.ds(base + c * _K, _K)], ss[b]
            )

        def wait_scatter(c, b):
            pltpu.make_async_copy(
                rows_v.at[b], out_hbm.at[pl.ds(base + c * _K, _K)], ss[b]
            ).wait()

        pltpu.sync_copy(idx_hbm.at[wid], idx_v)

        for b in range(_LOOK):
            fire_gather(b, b)

        def body(c0):
            for b in range(_NBUF):
                c = c0 + b
                bg = (b + _LOOK) % _NBUF
                cg = c + _LOOK

                @pl.when(cg < _CHUNKS)
                def _():
                    @pl.when(cg >= _NBUF)
                    def _():
                        wait_scatter(cg - _NBUF, bg)

                    fire_gather(cg, bg)

                wait_gather(c, b)
                fire_scatter(c, b)

        pl.loop(0, _CHUNKS, step=_NBUF)(body)

        for b in range(_NBUF):
            wait_scatter(_CHUNKS - _NBUF + b, b)

    return gather_kernel


_gather = _make_gather()


def kernel(x, tok_emb):
    idx = x.reshape(_NW, _CHUNKS, _K).astype(jnp.int32)
    out = _gather(idx, tok_emb)
    return out.reshape(BATCH, SEQ, HIDDEN_DIM)


# speedup vs baseline: 3.1410x; 3.1410x over previous
import functools

import jax
import jax.numpy as jnp
from jax import lax
from jax.experimental import pallas as pl
from jax.experimental.pallas import tpu as pltpu
from jax.experimental.pallas import tpu_sc as plsc

DICT_SIZE = 100000
HIDDEN_DIM = 128
BATCH = 4096
SEQ = 50

_NC = 2
_NS = 16
_NW = _NC * _NS

_N = BATCH * SEQ
_PER_W = _N // _NW
_K = 128
_CHUNKS = _PER_W // _K
_NBUF = 5
_LOOK = 1


def _make_gather():
    mesh = plsc.VectorSubcoreMesh(
        core_axis_name="c", subcore_axis_name="s",
        num_cores=_NC, num_subcores=_NS,
    )

    @functools.partial(
        pl.kernel,
        out_type=jax.ShapeDtypeStruct((_N, HIDDEN_DIM), jnp.float32),
        mesh=mesh,
        scratch_types=[
            pltpu.VMEM((_CHUNKS, _K), jnp.int32),
            pltpu.VMEM((_NBUF, _K, HIDDEN_DIM), jnp.float32),
            [pltpu.SemaphoreType.DMA] * _NBUF,
            [pltpu.SemaphoreType.DMA] * _NBUF,
        ],
    )
    def gather_kernel(idx_hbm, table_hbm, out_hbm, idx_v, rows_v, sg, ss):
        wid = lax.axis_index("s") * _NC + lax.axis_index("c")
        base = wid * _PER_W

        def fire_gather(c, b):
            pltpu.async_copy(table_hbm.at[idx_v.at[c]], rows_v.at[b], sg[b])

        def wait_gather(c, b):
            pltpu.make_async_copy(
                table_hbm.at[idx_v.at[c]], rows_v.at[b], sg[b]
            ).wait()

        def fire_scatter(c, b):
            pltpu.async_copy(
                rows_v.at[b], out_hbm.at[pl.ds(base + c * _K, _K)], ss[b]
            )

        def wait_scatter(c, b):
            pltpu.make_async_copy(
                rows_v.at[b], out_hbm.at[pl.ds(base + c * _K, _K)], ss[b]
            ).wait()

        pltpu.sync_copy(idx_hbm.at[wid], idx_v)

        for b in range(_LOOK):
            fire_gather(b, b)

        def body(c0):
            for b in range(_NBUF):
                c = c0 + b
                bg = (b + _LOOK) % _NBUF
                cg = c + _LOOK

                @pl.when(cg < _CHUNKS)
                def _():
                    @pl.when(cg >= _NBUF)
                    def _():
                        wait_scatter(cg - _NBUF, bg)

                    fire_gather(cg, bg)

                wait_gather(c, b)
                fire_scatter(c, b)

        pl.loop(0, _CHUNKS, step=_NBUF)(body)

        for b in range(_NBUF):
            wait_scatter(_CHUNKS - _NBUF + b, b)

    return gather_kernel


_gather = _make_gather()


def kernel(x, tok_emb):
    idx = x.T.reshape(_NW, _CHUNKS, _K).astype(jnp.int32)
    out = _gather(idx, tok_emb)
    return out.reshape(SEQ, BATCH, HIDDEN_DIM).transpose(1, 0, 2)
